# baseline (speedup 1.0000x reference)
import functools
import jax
import jax.numpy as jnp
from jax import lax
from jax.experimental import pallas as pl
from jax.experimental.pallas import tpu as pltpu
from jax.experimental.pallas import tpu_sc as plsc

N_BINS = 128
N_VERLET = 13
HALF = N_VERLET // 2
EMB = 256
N_T = 110
TP = 112
B = 1024
VOCAB = N_VERLET * N_VERLET
NC, NS = 2, 16
NW = NC * NS
ROWS = B // NW
L = 16
TB = 10


def _body(motx, moty, t0, t1, t2, bins_h, val_h, step_h, type_h, out_h,
          xv, yv, t0v, t1v, t2v, ti_v, binv, stepv, typev,
          xb_buf, yb_buf, tok0, tok1, ob0, ob1, sg0, sg1, sw0, sw1):
    wid = lax.axis_index("s") * NC + lax.axis_index("c")
    base = wid * ROWS

    pltpu.sync_copy(motx.at[pl.ds(base, ROWS)], xv)
    pltpu.sync_copy(moty.at[pl.ds(base, ROWS)], yv)
    pltpu.sync_copy(t0.at[pl.ds(base, ROWS)], t0v)
    pltpu.sync_copy(t1.at[pl.ds(base, ROWS)], t1v)
    pltpu.sync_copy(t2.at[pl.ds(base, ROWS)], t2v)
    pltpu.sync_copy(bins_h, binv)
    pltpu.sync_copy(step_h, stepv)
    pltpu.sync_copy(type_h, typev)

    iota = lax.iota(jnp.int32, L)

    for g in range(ROWS // L):
        a = t0v[pl.ds(L * g, L)]
        b = t1v[pl.ds(L * g, L)]
        c = t2v[pl.ds(L * g, L)]
        i01 = jnp.where(b > a, jnp.full((L,), 1, jnp.int32),
                        jnp.full((L,), 0, jnp.int32))
        v01 = jnp.maximum(a, b)
        ti = jnp.where(c > v01, jnp.full((L,), 2, jnp.int32), i01)
        ti_v[pl.ds(L * g, L)] = ti

    def searchsorted16(x):
        cnt = jnp.zeros((L,), jnp.int32)
        for s in (64, 32, 16, 8, 4, 2, 1):
            t = cnt + s
            bv = plsc.load_gather(binv, [t - 1])
            cnt = jnp.where(bv < x, t, cnt)
        return jnp.minimum(cnt, N_BINS - 1)

    def tokens_into(r, tokd):
        r = jnp.minimum(r, ROWS - 1)
        for k in range(TP // L):
            xb_buf[pl.ds(L * k, L)] = searchsorted16(xv[r, pl.ds(L * k, L)])
            yb_buf[pl.ds(L * k, L)] = searchsorted16(yv[r, pl.ds(L * k, L)])
        for k in range(TP // L):
            pidx = jnp.maximum(iota + (L * k - 1), 0)
            xp = plsc.load_gather(xb_buf, [pidx])
            yp = plsc.load_gather(yb_buf, [pidx])
            xc = xb_buf[pl.ds(L * k, L)]
            yc = yb_buf[pl.ds(L * k, L)]
            dx = jnp.clip(xc - xp, -HALF, HALF) + HALF
            dy = jnp.clip(yc - yp, -HALF, HALF) + HALF
            tokd[pl.ds(L * k, L)] = dx * N_VERLET + dy

    def adds_into(r, ob):
        ti16 = plsc.load_gather(ti_v, [jnp.full((L,), r, jnp.int32)])
        tvecs = [plsc.load_gather(typev, [ti16 * EMB + (L * c + iota)])
                 for c in range(EMB // L)]

        def blk(i, _):
            tbase = i * TB
            sbase = i * (TB * EMB)
            for j in range(TB):
                for c in range(EMB // L):
                    sv = stepv[pl.ds(sbase + (j * EMB + L * c), L)]
                    plsc.addupdate(ob.at[tbase + j, pl.ds(L * c, L)],
                                   sv + tvecs[c])
            return 0

        lax.fori_loop(0, N_T // TB, blk, 0)

    def gather_of(tokd, ob, sg):
        return pltpu.make_async_copy(val_h.at[tokd], ob, sg)

    def write_of(r, ob, sw):
        return pltpu.make_async_copy(
            ob.at[pl.ds(0, N_T)], out_h.at[pl.ds((base + r) * N_T, N_T)], sw)

    tokens_into(0, tok0)
    gather_of(tok0, ob0, sg0).start()
    pltpu.make_async_copy(out_h.at[pl.ds(0, N_T)],
                          ob1.at[pl.ds(0, N_T)], sw1).start()

    def pair(i, carry):
        s0 = 2 * i
        gather_of(tok0, ob0, sg0).wait()
        tokens_into(s0 + 1, tok1)
        write_of(0, ob1, sw1).wait()
        gather_of(tok1, ob1, sg1).start()
        adds_into(s0, ob0)
        write_of(s0, ob0, sw0).start()
        gather_of(tok1, ob1, sg1).wait()
        tokens_into(s0 + 2, tok0)
        write_of(s0, ob0, sw0).wait()

        @pl.when(i < (ROWS // 2 - 1))
        def _():
            gather_of(tok0, ob0, sg0).start()

        adds_into(s0 + 1, ob1)
        write_of(s0 + 1, ob1, sw1).start()
        return carry

    lax.fori_loop(0, ROWS // 2, pair, 0)
    write_of(ROWS - 1, ob1, sw1).wait()


@jax.jit
def _run(motx, moty, t0, t1, t2, pos_bins, val_emb, step_flat, type_flat):
    mesh = plsc.VectorSubcoreMesh(core_axis_name="c", subcore_axis_name="s")
    f = pl.kernel(
        _body,
        out_type=jax.ShapeDtypeStruct((B * N_T, EMB), jnp.float32),
        mesh=mesh,
        compiler_params=pltpu.CompilerParams(use_tc_tiling_on_sc=False,
                                             needs_layout_passes=False),
        scratch_types=[
            pltpu.VMEM((ROWS, TP), jnp.float32),
            pltpu.VMEM((ROWS, TP), jnp.float32),
            pltpu.VMEM((ROWS,), jnp.float32),
            pltpu.VMEM((ROWS,), jnp.float32),
            pltpu.VMEM((ROWS,), jnp.float32),
            pltpu.VMEM((ROWS,), jnp.int32),
            pltpu.VMEM((N_BINS,), jnp.float32),
            pltpu.VMEM((N_T * EMB,), jnp.float32),
            pltpu.VMEM((3 * EMB,), jnp.float32),
            pltpu.VMEM((TP,), jnp.int32),
            pltpu.VMEM((TP,), jnp.int32),
            pltpu.VMEM((TP,), jnp.int32),
            pltpu.VMEM((TP,), jnp.int32),
            pltpu.VMEM((TP, EMB), jnp.float32),
            pltpu.VMEM((TP, EMB), jnp.float32),
            pltpu.SemaphoreType.DMA,
            pltpu.SemaphoreType.DMA,
            pltpu.SemaphoreType.DMA,
            pltpu.SemaphoreType.DMA,
        ],
    )
    return f(motx, moty, t0, t1, t2, pos_bins, val_emb, step_flat, type_flat)


def kernel(motion_tokens, target_types, fused_emb, fused_emb_invalid,
           val_emb, step_emb, type_emb):
    pos_bins = jnp.linspace(-4.0, 4.0, N_BINS)
    motx = jnp.pad(motion_tokens[:, :, 0], ((0, 0), (0, TP - N_T)))
    moty = jnp.pad(motion_tokens[:, :, 1], ((0, 0), (0, TP - N_T)))
    t0 = target_types[:, 0]
    t1 = target_types[:, 1]
    t2 = target_types[:, 2]
    return _run(motx, moty, t0, t1, t2, pos_bins, val_emb,
                step_emb.reshape(-1), type_emb.reshape(-1))


# speedup vs baseline: 2.7688x; 2.7688x over previous
import functools
import jax
import jax.numpy as jnp
from jax import lax
from jax.experimental import pallas as pl
from jax.experimental.pallas import tpu as pltpu
from jax.experimental.pallas import tpu_sc as plsc

N_BINS = 128
N_VERLET = 13
HALF = N_VERLET // 2
EMB = 256
N_T = 110
TP = 112
B = 1024
VOCAB = N_VERLET * N_VERLET
NC, NS = 2, 16
NW = NC * NS
ROWS = B // NW
L = 16
TB = 10


def _body(motx, moty, t0, t1, t2, bins_h, val_h, step_h, type_h, out_h,
          xv, yv, t0v, t1v, t2v, ti_v, binv, stepv, typev,
          xb_buf, yb_buf, tok0, tok1, ob0, ob1, sg0, sg1, sw0, sw1):
    wid = lax.axis_index("s") * NC + lax.axis_index("c")
    base = wid * ROWS

    pltpu.sync_copy(motx.at[pl.ds(base, ROWS)], xv)
    pltpu.sync_copy(moty.at[pl.ds(base, ROWS)], yv)
    pltpu.sync_copy(t0.at[pl.ds(base, ROWS)], t0v)
    pltpu.sync_copy(t1.at[pl.ds(base, ROWS)], t1v)
    pltpu.sync_copy(t2.at[pl.ds(base, ROWS)], t2v)
    pltpu.sync_copy(bins_h, binv)
    pltpu.sync_copy(step_h, stepv)
    pltpu.sync_copy(type_h, typev)

    iota = lax.iota(jnp.int32, L)

    for g in range(ROWS // L):
        a = t0v[pl.ds(L * g, L)]
        b = t1v[pl.ds(L * g, L)]
        c = t2v[pl.ds(L * g, L)]
        i01 = jnp.where(b > a, jnp.full((L,), 1, jnp.int32),
                        jnp.full((L,), 0, jnp.int32))
        v01 = jnp.maximum(a, b)
        ti = jnp.where(c > v01, jnp.full((L,), 2, jnp.int32), i01)
        ti_v[pl.ds(L * g, L)] = ti

    def searchsorted16(x):
        cnt = jnp.zeros((L,), jnp.int32)
        for s in (64, 32, 16, 8, 4, 2, 1):
            t = cnt + s
            bv = plsc.load_gather(binv, [t - 1])
            cnt = jnp.where(bv < x, t, cnt)
        return jnp.minimum(cnt, N_BINS - 1)

    def tokens_into(r, tokd):
        r = jnp.minimum(r, ROWS - 1)
        for k in range(TP // L):
            xb_buf[pl.ds(L * k, L)] = searchsorted16(xv[r, pl.ds(L * k, L)])
            yb_buf[pl.ds(L * k, L)] = searchsorted16(yv[r, pl.ds(L * k, L)])
        for k in range(TP // L):
            pidx = jnp.maximum(iota + (L * k - 1), 0)
            xp = plsc.load_gather(xb_buf, [pidx])
            yp = plsc.load_gather(yb_buf, [pidx])
            xc = xb_buf[pl.ds(L * k, L)]
            yc = yb_buf[pl.ds(L * k, L)]
            dx = jnp.clip(xc - xp, -HALF, HALF) + HALF
            dy = jnp.clip(yc - yp, -HALF, HALF) + HALF
            tokd[pl.ds(L * k, L)] = dx * N_VERLET + dy + wid * VOCAB

    def adds_into(r, ob):
        ti16 = plsc.load_gather(ti_v, [jnp.full((L,), r, jnp.int32)])
        tvecs = [plsc.load_gather(typev, [ti16 * EMB + (L * c + iota)])
                 for c in range(EMB // L)]

        def blk(i, _):
            tbase = i * TB
            sbase = i * (TB * EMB)
            for j in range(TB):
                for c in range(EMB // L):
                    sv = stepv[pl.ds(sbase + (j * EMB + L * c), L)]
                    plsc.addupdate(ob.at[tbase + j, pl.ds(L * c, L)],
                                   sv + tvecs[c])
            return 0

        lax.fori_loop(0, N_T // TB, blk, 0)

    def gather_of(tokd, ob, sg):
        return pltpu.make_async_copy(val_h.at[tokd], ob, sg)

    def write_of(r, ob, sw):
        return pltpu.make_async_copy(
            ob.at[pl.ds(0, N_T)], out_h.at[pl.ds((base + r) * N_T, N_T)], sw)

    tokens_into(0, tok0)
    gather_of(tok0, ob0, sg0).start()
    pltpu.make_async_copy(out_h.at[pl.ds(0, N_T)],
                          ob1.at[pl.ds(0, N_T)], sw1).start()

    def pair(i, carry):
        s0 = 2 * i
        gather_of(tok0, ob0, sg0).wait()
        tokens_into(s0 + 1, tok1)
        write_of(0, ob1, sw1).wait()
        gather_of(tok1, ob1, sg1).start()
        adds_into(s0, ob0)
        write_of(s0, ob0, sw0).start()
        gather_of(tok1, ob1, sg1).wait()
        tokens_into(s0 + 2, tok0)
        write_of(s0, ob0, sw0).wait()

        @pl.when(i < (ROWS // 2 - 1))
        def _():
            gather_of(tok0, ob0, sg0).start()

        adds_into(s0 + 1, ob1)
        write_of(s0 + 1, ob1, sw1).start()
        return carry

    lax.fori_loop(0, ROWS // 2, pair, 0)
    write_of(ROWS - 1, ob1, sw1).wait()


@jax.jit
def _run(motx, moty, t0, t1, t2, pos_bins, val_emb, step_flat, type_flat):
    mesh = plsc.VectorSubcoreMesh(core_axis_name="c", subcore_axis_name="s")
    f = pl.kernel(
        _body,
        out_type=jax.ShapeDtypeStruct((B * N_T, EMB), jnp.float32),
        mesh=mesh,
        compiler_params=pltpu.CompilerParams(use_tc_tiling_on_sc=False,
                                             needs_layout_passes=False),
        scratch_types=[
            pltpu.VMEM((ROWS, TP), jnp.float32),
            pltpu.VMEM((ROWS, TP), jnp.float32),
            pltpu.VMEM((ROWS,), jnp.float32),
            pltpu.VMEM((ROWS,), jnp.float32),
            pltpu.VMEM((ROWS,), jnp.float32),
            pltpu.VMEM((ROWS,), jnp.int32),
            pltpu.VMEM((N_BINS,), jnp.float32),
            pltpu.VMEM((N_T * EMB,), jnp.float32),
            pltpu.VMEM((3 * EMB,), jnp.float32),
            pltpu.VMEM((TP,), jnp.int32),
            pltpu.VMEM((TP,), jnp.int32),
            pltpu.VMEM((TP,), jnp.int32),
            pltpu.VMEM((TP,), jnp.int32),
            pltpu.VMEM((TP, EMB), jnp.float32),
            pltpu.VMEM((TP, EMB), jnp.float32),
            pltpu.SemaphoreType.DMA,
            pltpu.SemaphoreType.DMA,
            pltpu.SemaphoreType.DMA,
            pltpu.SemaphoreType.DMA,
        ],
    )
    return f(motx, moty, t0, t1, t2, pos_bins, val_emb, step_flat, type_flat)


def kernel(motion_tokens, target_types, fused_emb, fused_emb_invalid,
           val_emb, step_emb, type_emb):
    pos_bins = jnp.linspace(-4.0, 4.0, N_BINS)
    motx = jnp.pad(motion_tokens[:, :, 0], ((0, 0), (0, TP - N_T)))
    moty = jnp.pad(motion_tokens[:, :, 1], ((0, 0), (0, TP - N_T)))
    t0 = target_types[:, 0]
    t1 = target_types[:, 1]
    t2 = target_types[:, 2]
    val_rep = jnp.tile(val_emb, (NW, 1))
    return _run(motx, moty, t0, t1, t2, pos_bins, val_rep,
                step_emb.reshape(-1), type_emb.reshape(-1))


# speedup vs baseline: 2.8248x; 1.0202x over previous
import functools
import jax
import jax.numpy as jnp
from jax import lax
from jax.experimental import pallas as pl
from jax.experimental.pallas import tpu as pltpu
from jax.experimental.pallas import tpu_sc as plsc

N_BINS = 128
N_VERLET = 13
HALF = N_VERLET // 2
EMB = 256
N_T = 110
TP = 112
B = 1024
VOCAB = N_VERLET * N_VERLET
NC, NS = 2, 16
NW = NC * NS
ROWS = B // NW
L = 16
TB = 10


def _body(motx, moty, t0, t1, t2, bins_h, val_h, step_h, type_h, out_h,
          xv, yv, t0v, t1v, t2v, ti_v, binv, stepv, typev,
          xb_buf, yb_buf, tok0, tok1, ob0, ob1, val_sh, sg0, sg1, sw0, sw1):
    sid = lax.axis_index("s")
    wid = sid * NC + lax.axis_index("c")
    base = wid * ROWS

    slot = lax.div(sid, 2)

    @pl.when(lax.rem(sid, 2) == 0)
    def _():
        pltpu.sync_copy(val_h, val_sh.at[pl.ds(slot * VOCAB, VOCAB)])

    plsc.subcore_barrier()

    pltpu.sync_copy(motx.at[pl.ds(base, ROWS)], xv)
    pltpu.sync_copy(moty.at[pl.ds(base, ROWS)], yv)
    pltpu.sync_copy(t0.at[pl.ds(base, ROWS)], t0v)
    pltpu.sync_copy(t1.at[pl.ds(base, ROWS)], t1v)
    pltpu.sync_copy(t2.at[pl.ds(base, ROWS)], t2v)
    pltpu.sync_copy(bins_h, binv)
    pltpu.sync_copy(step_h, stepv)
    pltpu.sync_copy(type_h, typev)

    iota = lax.iota(jnp.int32, L)

    for g in range(ROWS // L):
        a = t0v[pl.ds(L * g, L)]
        b = t1v[pl.ds(L * g, L)]
        c = t2v[pl.ds(L * g, L)]
        i01 = jnp.where(b > a, jnp.full((L,), 1, jnp.int32),
                        jnp.full((L,), 0, jnp.int32))
        v01 = jnp.maximum(a, b)
        ti = jnp.where(c > v01, jnp.full((L,), 2, jnp.int32), i01)
        ti_v[pl.ds(L * g, L)] = ti

    def searchsorted16(x):
        cnt = jnp.zeros((L,), jnp.int32)
        for s in (64, 32, 16, 8, 4, 2, 1):
            t = cnt + s
            bv = plsc.load_gather(binv, [t - 1])
            cnt = jnp.where(bv < x, t, cnt)
        return jnp.minimum(cnt, N_BINS - 1)

    def tokens_into(r, tokd):
        r = jnp.minimum(r, ROWS - 1)
        for k in range(TP // L):
            xb_buf[pl.ds(L * k, L)] = searchsorted16(xv[r, pl.ds(L * k, L)])
            yb_buf[pl.ds(L * k, L)] = searchsorted16(yv[r, pl.ds(L * k, L)])
        for k in range(TP // L):
            pidx = jnp.maximum(iota + (L * k - 1), 0)
            xp = plsc.load_gather(xb_buf, [pidx])
            yp = plsc.load_gather(yb_buf, [pidx])
            xc = xb_buf[pl.ds(L * k, L)]
            yc = yb_buf[pl.ds(L * k, L)]
            dx = jnp.clip(xc - xp, -HALF, HALF) + HALF
            dy = jnp.clip(yc - yp, -HALF, HALF) + HALF
            tokd[pl.ds(L * k, L)] = dx * N_VERLET + dy + slot * VOCAB

    def adds_into(r, ob):
        ti16 = plsc.load_gather(ti_v, [jnp.full((L,), r, jnp.int32)])
        tvecs = [plsc.load_gather(typev, [ti16 * EMB + (L * c + iota)])
                 for c in range(EMB // L)]

        def blk(i, _):
            tbase = i * TB
            sbase = i * (TB * EMB)
            for j in range(TB):
                for c in range(EMB // L):
                    sv = stepv[pl.ds(sbase + (j * EMB + L * c), L)]
                    plsc.addupdate(ob.at[tbase + j, pl.ds(L * c, L)],
                                   sv + tvecs[c])
            return 0

        lax.fori_loop(0, N_T // TB, blk, 0)

    def gather_of(tokd, ob, sg):
        return pltpu.make_async_copy(val_sh.at[tokd], ob, sg)

    def write_of(r, ob, sw):
        return pltpu.make_async_copy(
            ob.at[pl.ds(0, N_T)], out_h.at[pl.ds((base + r) * N_T, N_T)], sw)

    tokens_into(0, tok0)
    gather_of(tok0, ob0, sg0).start()
    pltpu.make_async_copy(out_h.at[pl.ds(0, N_T)],
                          ob1.at[pl.ds(0, N_T)], sw1).start()

    def pair(i, carry):
        s0 = 2 * i
        gather_of(tok0, ob0, sg0).wait()
        tokens_into(s0 + 1, tok1)
        write_of(0, ob1, sw1).wait()
        gather_of(tok1, ob1, sg1).start()
        adds_into(s0, ob0)
        write_of(s0, ob0, sw0).start()
        gather_of(tok1, ob1, sg1).wait()
        tokens_into(s0 + 2, tok0)
        write_of(s0, ob0, sw0).wait()

        @pl.when(i < (ROWS // 2 - 1))
        def _():
            gather_of(tok0, ob0, sg0).start()

        adds_into(s0 + 1, ob1)
        write_of(s0 + 1, ob1, sw1).start()
        return carry

    lax.fori_loop(0, ROWS // 2, pair, 0)
    write_of(ROWS - 1, ob1, sw1).wait()


@jax.jit
def _run(motx, moty, t0, t1, t2, pos_bins, val_emb, step_flat, type_flat):
    mesh = plsc.VectorSubcoreMesh(core_axis_name="c", subcore_axis_name="s")
    f = pl.kernel(
        _body,
        out_type=jax.ShapeDtypeStruct((B * N_T, EMB), jnp.float32),
        mesh=mesh,
        compiler_params=pltpu.CompilerParams(use_tc_tiling_on_sc=False,
                                             needs_layout_passes=False),
        scratch_types=[
            pltpu.VMEM((ROWS, TP), jnp.float32),
            pltpu.VMEM((ROWS, TP), jnp.float32),
            pltpu.VMEM((ROWS,), jnp.float32),
            pltpu.VMEM((ROWS,), jnp.float32),
            pltpu.VMEM((ROWS,), jnp.float32),
            pltpu.VMEM((ROWS,), jnp.int32),
            pltpu.VMEM((N_BINS,), jnp.float32),
            pltpu.VMEM((N_T * EMB,), jnp.float32),
            pltpu.VMEM((3 * EMB,), jnp.float32),
            pltpu.VMEM((TP,), jnp.int32),
            pltpu.VMEM((TP,), jnp.int32),
            pltpu.VMEM((TP,), jnp.int32),
            pltpu.VMEM((TP,), jnp.int32),
            pltpu.VMEM((TP, EMB), jnp.float32),
            pltpu.VMEM((TP, EMB), jnp.float32),
            pltpu.VMEM_SHARED((NS // 2 * VOCAB, EMB), jnp.float32),
            pltpu.SemaphoreType.DMA,
            pltpu.SemaphoreType.DMA,
            pltpu.SemaphoreType.DMA,
            pltpu.SemaphoreType.DMA,
        ],
    )
    return f(motx, moty, t0, t1, t2, pos_bins, val_emb, step_flat, type_flat)


def kernel(motion_tokens, target_types, fused_emb, fused_emb_invalid,
           val_emb, step_emb, type_emb):
    pos_bins = jnp.linspace(-4.0, 4.0, N_BINS)
    motx = jnp.pad(motion_tokens[:, :, 0], ((0, 0), (0, TP - N_T)))
    moty = jnp.pad(motion_tokens[:, :, 1], ((0, 0), (0, TP - N_T)))
    t0 = target_types[:, 0]
    t1 = target_types[:, 1]
    t2 = target_types[:, 2]
    return _run(motx, moty, t0, t1, t2, pos_bins, val_emb,
                step_emb.reshape(-1), type_emb.reshape(-1))
